# baseline (device time: 20929 ns/iter reference)
import jax
import jax.numpy as jnp
from jax import lax
from jax.experimental import pallas as pl
from jax.experimental.pallas import tpu as pltpu


def kernel(x, dy, gamma):
    m, d = x.shape

    def body(x_ref, dy_ref, out_ref, send_buf, recv_buf, send_sem, recv_sem):
        my_x = lax.axis_index("x")
        my_y = lax.axis_index("y")
        my_z = lax.axis_index("z")
        partner = (1 - my_x, my_y, my_z)

        xv = x_ref[:, :]
        dyv = dy_ref[:, :]
        mu = jnp.mean(xv, axis=1, keepdims=True)
        cen = xv - mu
        var = jnp.mean(cen * cen, axis=1, keepdims=True)
        xhat = cen * lax.rsqrt(var + 1e-5)
        send_buf[0, :] = jnp.sum(dyv * xhat, axis=0)
        send_buf[1, :] = jnp.sum(dyv, axis=0)

        barrier_sem = pltpu.get_barrier_semaphore()
        pl.semaphore_signal(
            barrier_sem, inc=1,
            device_id=partner, device_id_type=pl.DeviceIdType.MESH,
        )
        pl.semaphore_wait(barrier_sem, 1)

        rdma = pltpu.make_async_remote_copy(
            src_ref=send_buf,
            dst_ref=recv_buf,
            send_sem=send_sem,
            recv_sem=recv_sem,
            device_id=partner,
            device_id_type=pl.DeviceIdType.MESH,
        )
        rdma.start()
        rdma.wait()

        out_ref[:, :] = send_buf[:, :] + recv_buf[:, :]

    return pl.pallas_call(
        body,
        out_shape=jax.ShapeDtypeStruct((2, d), jnp.float32),
        in_specs=[
            pl.BlockSpec(memory_space=pltpu.VMEM),
            pl.BlockSpec(memory_space=pltpu.VMEM),
        ],
        out_specs=pl.BlockSpec(memory_space=pltpu.VMEM),
        scratch_shapes=[
            pltpu.VMEM((2, d), jnp.float32),
            pltpu.VMEM((2, d), jnp.float32),
            pltpu.SemaphoreType.DMA,
            pltpu.SemaphoreType.DMA,
        ],
        compiler_params=pltpu.CompilerParams(collective_id=0),
    )(x, dy)


# device time: 20610 ns/iter; 1.0155x vs baseline; 1.0155x over previous
import jax
import jax.numpy as jnp
from jax import lax
from jax.experimental import pallas as pl
from jax.experimental.pallas import tpu as pltpu

BM = 256


def kernel(x, dy, gamma):
    m, d = x.shape
    nb = m // BM

    def body(x_ref, dy_ref, out_ref, acc, recv_buf, send_sem, recv_sem):
        i = pl.program_id(0)

        xv = x_ref[:, :]
        dyv = dy_ref[:, :]
        mu = jnp.mean(xv, axis=1, keepdims=True)
        cen = xv - mu
        var = jnp.mean(cen * cen, axis=1, keepdims=True)
        xhat = cen * lax.rsqrt(var + 1e-5)
        pg = jnp.sum(dyv * xhat, axis=0)
        pb = jnp.sum(dyv, axis=0)

        @pl.when(i == 0)
        def _():
            acc[0, :] = pg
            acc[1, :] = pb

        @pl.when(i > 0)
        def _():
            acc[0, :] += pg
            acc[1, :] += pb

        @pl.when(i == nb - 1)
        def _():
            my_x = lax.axis_index("x")
            my_y = lax.axis_index("y")
            my_z = lax.axis_index("z")
            partner = (1 - my_x, my_y, my_z)

            barrier_sem = pltpu.get_barrier_semaphore()
            pl.semaphore_signal(
                barrier_sem, inc=1,
                device_id=partner, device_id_type=pl.DeviceIdType.MESH,
            )
            pl.semaphore_wait(barrier_sem, 1)

            rdma = pltpu.make_async_remote_copy(
                src_ref=acc,
                dst_ref=recv_buf,
                send_sem=send_sem,
                recv_sem=recv_sem,
                device_id=partner,
                device_id_type=pl.DeviceIdType.MESH,
            )
            rdma.start()
            rdma.wait()

            out_ref[:, :] = acc[:, :] + recv_buf[:, :]

    return pl.pallas_call(
        body,
        grid=(nb,),
        out_shape=jax.ShapeDtypeStruct((2, d), jnp.float32),
        in_specs=[
            pl.BlockSpec((BM, d), lambda i: (i, 0)),
            pl.BlockSpec((BM, d), lambda i: (i, 0)),
        ],
        out_specs=pl.BlockSpec((2, d), lambda i: (0, 0)),
        scratch_shapes=[
            pltpu.VMEM((2, d), jnp.float32),
            pltpu.VMEM((2, d), jnp.float32),
            pltpu.SemaphoreType.DMA,
            pltpu.SemaphoreType.DMA,
        ],
        compiler_params=pltpu.CompilerParams(collective_id=0),
    )(x, dy)


# device time: 20580 ns/iter; 1.0170x vs baseline; 1.0015x over previous
import jax
import jax.numpy as jnp
from jax import lax
from jax.experimental import pallas as pl
from jax.experimental.pallas import tpu as pltpu

BM = 256


def kernel(x, dy, gamma):
    m, d = x.shape
    nb = m // BM

    def body(x_ref, dy_ref, out_ref, acc, recv_buf, send_sem, recv_sem):
        i = pl.program_id(0)

        xv = x_ref[:, :]
        dyv = dy_ref[:, :]
        s1 = jnp.sum(xv, axis=1)
        s2 = jnp.sum(xv * xv, axis=1)
        mu = s1 / d
        var = s2 / d - mu * mu
        rstd = lax.rsqrt(var + 1e-5)
        t = xv * dyv
        r1 = lax.dot_general(
            rstd[None, :], t,
            dimension_numbers=(((1,), (0,)), ((), ())),
            preferred_element_type=jnp.float32,
        )
        w2 = jnp.stack([-rstd * mu, jnp.ones_like(rstd)])
        r2 = lax.dot_general(
            w2, dyv,
            dimension_numbers=(((1,), (0,)), ((), ())),
            preferred_element_type=jnp.float32,
        )
        pg = r1[0] + r2[0]
        pb = r2[1]

        @pl.when(i == 0)
        def _():
            acc[0, :] = pg
            acc[1, :] = pb

        @pl.when(i > 0)
        def _():
            acc[0, :] += pg
            acc[1, :] += pb

        @pl.when(i == nb - 1)
        def _():
            my_x = lax.axis_index("x")
            my_y = lax.axis_index("y")
            my_z = lax.axis_index("z")
            partner = (1 - my_x, my_y, my_z)

            barrier_sem = pltpu.get_barrier_semaphore()
            pl.semaphore_signal(
                barrier_sem, inc=1,
                device_id=partner, device_id_type=pl.DeviceIdType.MESH,
            )
            pl.semaphore_wait(barrier_sem, 1)

            rdma = pltpu.make_async_remote_copy(
                src_ref=acc,
                dst_ref=recv_buf,
                send_sem=send_sem,
                recv_sem=recv_sem,
                device_id=partner,
                device_id_type=pl.DeviceIdType.MESH,
            )
            rdma.start()
            rdma.wait()

            out_ref[:, :] = acc[:, :] + recv_buf[:, :]

    return pl.pallas_call(
        body,
        grid=(nb,),
        out_shape=jax.ShapeDtypeStruct((2, d), jnp.float32),
        in_specs=[
            pl.BlockSpec((BM, d), lambda i: (i, 0)),
            pl.BlockSpec((BM, d), lambda i: (i, 0)),
        ],
        out_specs=pl.BlockSpec((2, d), lambda i: (0, 0)),
        scratch_shapes=[
            pltpu.VMEM((2, d), jnp.float32),
            pltpu.VMEM((2, d), jnp.float32),
            pltpu.SemaphoreType.DMA,
            pltpu.SemaphoreType.DMA,
        ],
        compiler_params=pltpu.CompilerParams(collective_id=0),
    )(x, dy)


# device time: 20555 ns/iter; 1.0182x vs baseline; 1.0012x over previous
import jax
import jax.numpy as jnp
from jax import lax
from jax.experimental import pallas as pl
from jax.experimental.pallas import tpu as pltpu

BM = 256


def kernel(x, dy, gamma):
    m, d = x.shape
    nb = m // BM

    def body(x_ref, dy_ref, out_ref, acc, recv_buf, send_sem, recv_sem):
        i = pl.program_id(0)

        my_x = lax.axis_index("x")
        my_y = lax.axis_index("y")
        my_z = lax.axis_index("z")
        partner = (1 - my_x, my_y, my_z)

        @pl.when(i == 0)
        def _():
            barrier_sem = pltpu.get_barrier_semaphore()
            pl.semaphore_signal(
                barrier_sem, inc=1,
                device_id=partner, device_id_type=pl.DeviceIdType.MESH,
            )
            pl.semaphore_wait(barrier_sem, 1)

        xv = x_ref[:, :]
        dyv = dy_ref[:, :]
        mu = jnp.mean(xv, axis=1, keepdims=True)
        cen = xv - mu
        var = jnp.mean(cen * cen, axis=1, keepdims=True)
        xhat = cen * lax.rsqrt(var + 1e-5)
        pg = jnp.sum(dyv * xhat, axis=0)
        pb = jnp.sum(dyv, axis=0)

        @pl.when(i == 0)
        def _():
            acc[0, :] = pg
            acc[1, :] = pb

        @pl.when(i > 0)
        def _():
            acc[0, :] += pg
            acc[1, :] += pb

        @pl.when(i == nb - 1)
        def _():
            rdma = pltpu.make_async_remote_copy(
                src_ref=acc,
                dst_ref=recv_buf,
                send_sem=send_sem,
                recv_sem=recv_sem,
                device_id=partner,
                device_id_type=pl.DeviceIdType.MESH,
            )
            rdma.start()
            rdma.wait()

            out_ref[:, :] = acc[:, :] + recv_buf[:, :]

    return pl.pallas_call(
        body,
        grid=(nb,),
        out_shape=jax.ShapeDtypeStruct((2, d), jnp.float32),
        in_specs=[
            pl.BlockSpec((BM, d), lambda i: (i, 0)),
            pl.BlockSpec((BM, d), lambda i: (i, 0)),
        ],
        out_specs=pl.BlockSpec((2, d), lambda i: (0, 0)),
        scratch_shapes=[
            pltpu.VMEM((2, d), jnp.float32),
            pltpu.VMEM((2, d), jnp.float32),
            pltpu.SemaphoreType.DMA,
            pltpu.SemaphoreType.DMA,
        ],
        compiler_params=pltpu.CompilerParams(collective_id=0),
    )(x, dy)
